# Initial kernel scaffold; baseline (speedup 1.0000x reference)
import jax
import jax.numpy as jnp
from jax.experimental import pallas as pl


def kernel(preds_S, preds_T, gt_labels):
    raise NotImplementedError("write your pallas kernel here")


# speedup vs baseline: 5.9635x; 5.9635x over previous
import functools

import jax
import jax.numpy as jnp
from jax import lax
from jax.experimental import pallas as pl
from jax.experimental.pallas import tpu as pltpu
from jax.experimental.pallas import tpu_sc as plsc

_TAU = 1.0
_C = 14
_B = 4
_P = 512 * 512
_BLK = 8192

_NSUB = 16
_PER = (_B * _P) // _NSUB
_CHUNK = 16384
_NCH = _PER // _CHUNK
_VECS = _CHUNK // 16


def _kl_body(s_ref, t_ref, out_ref):
    s = s_ref[0]
    t = t_ref[0]
    ms = jnp.max(s, axis=0, keepdims=True)
    mt = jnp.max(t, axis=0, keepdims=True)
    es = jnp.exp(s - ms)
    et = jnp.exp(t - mt)
    zs = jnp.sum(es, axis=0, keepdims=True)
    zt = jnp.sum(et, axis=0, keepdims=True)
    num = jnp.sum(et * (t - s), axis=0, keepdims=True)
    out_ref[0] = num / zt + (ms - mt) + jnp.log(zs) - jnp.log(zt)


def _seg_body(kl_hbm, gt_hbm, out_hbm, klv, gtv, acc2, cnt2, svb, cvb,
              outv, sh_sums, sh_cnts):
    sid = lax.axis_index("s")
    base = sid * _PER
    lane = lax.iota(jnp.int32, 16)
    zero16 = jnp.zeros((16,), jnp.float32)
    ones16 = jnp.ones((16,), jnp.float32)

    lane16 = lane * 16
    for r in range(16):
        acc2[pl.ds(r * 16, 16)] = zero16
        cnt2[pl.ds(r * 16, 16)] = zero16

    for ch in range(_NCH):
        start = base + ch * _CHUNK
        pltpu.sync_copy(kl_hbm.at[pl.ds(start, _CHUNK)], klv)
        pltpu.sync_copy(gt_hbm.at[pl.ds(start, _CHUNK)], gtv)

        def body(i, carry):
            off = i * 16
            kv = klv[pl.ds(off, 16)]
            gv = gtv[pl.ds(off, 16)]
            plsc.addupdate_scatter(acc2, [lane16 + gv], kv)
            plsc.addupdate_scatter(cnt2, [lane16 + gv], ones16)
            return carry

        lax.fori_loop(0, _VECS, body, 0)

    sv = acc2[pl.ds(0, 16)]
    cv = cnt2[pl.ds(0, 16)]
    for r in range(1, 16):
        sv = sv + acc2[pl.ds(r * 16, 16)]
        cv = cv + cnt2[pl.ds(r * 16, 16)]
    svb[...] = sv
    cvb[...] = cv

    pltpu.sync_copy(svb, sh_sums.at[pl.ds(sid * 16, 16)])
    pltpu.sync_copy(cvb, sh_cnts.at[pl.ds(sid * 16, 16)])
    plsc.subcore_barrier()

    @pl.when(sid == 0)
    def _():
        pltpu.sync_copy(sh_sums, acc2)
        pltpu.sync_copy(sh_cnts, cnt2)
        cls_mask = (lane >= 1) & (lane < _C)
        spb = _NSUB // _B
        total = jnp.float32(0.0)
        for bb in range(_B):
            s = acc2[pl.ds(spb * bb * 16, 16)]
            c = cnt2[pl.ds(spb * bb * 16, 16)]
            for r in range(1, spb):
                s = s + acc2[pl.ds((spb * bb + r) * 16, 16)]
                c = c + cnt2[pl.ds((spb * bb + r) * 16, 16)]
            per = jnp.where(c > 0.0, s / (_C * jnp.maximum(c, 1.0)), 0.0)
            per = jnp.where(cls_mask, per, 0.0)
            total = total + jnp.sum(per)
        outv[...] = ones16 * total
        pltpu.sync_copy(outv, out_hbm)


@functools.cache
def _seg_reduce():
    @functools.partial(
        pl.kernel,
        out_type=jax.ShapeDtypeStruct((16,), jnp.float32),
        mesh=plsc.VectorSubcoreMesh(core_axis_name="c", subcore_axis_name="s",
                                    num_cores=1),
        compiler_params=pltpu.CompilerParams(needs_layout_passes=False),
        scratch_types=[
            pltpu.VMEM((_CHUNK,), jnp.float32),
            pltpu.VMEM((_CHUNK,), jnp.int32),
            pltpu.VMEM((256,), jnp.float32),
            pltpu.VMEM((256,), jnp.float32),
            pltpu.VMEM((16,), jnp.float32),
            pltpu.VMEM((16,), jnp.float32),
            pltpu.VMEM((16,), jnp.float32),
            pltpu.VMEM_SHARED((_NSUB * 16,), jnp.float32),
            pltpu.VMEM_SHARED((_NSUB * 16,), jnp.float32),
        ],
    )
    def seg(kl_hbm, gt_hbm, out_hbm, *scratch):
        _seg_body(kl_hbm, gt_hbm, out_hbm, *scratch)

    return seg


def kernel(preds_S, preds_T, gt_labels):
    s3 = preds_S.reshape(_B, _C, _P)
    t3 = preds_T.reshape(_B, _C, _P)
    kl_pix = pl.pallas_call(
        _kl_body,
        grid=(_B, _P // _BLK),
        in_specs=[
            pl.BlockSpec((1, _C, _BLK), lambda b, j: (b, 0, j)),
            pl.BlockSpec((1, _C, _BLK), lambda b, j: (b, 0, j)),
        ],
        out_specs=pl.BlockSpec((1, 1, _BLK), lambda b, j: (b, 0, j)),
        out_shape=jax.ShapeDtypeStruct((_B, 1, _P), jnp.float32),
    )(s3, t3)

    gt_flat = gt_labels.reshape(_B * _P).astype(jnp.int32)
    out_vec = _seg_reduce()(kl_pix.reshape(_B * _P), gt_flat)
    return out_vec[0] * (_TAU * _TAU)


# speedup vs baseline: 11.8311x; 1.9839x over previous
import functools

import jax
import jax.numpy as jnp
from jax import lax
from jax.experimental import pallas as pl
from jax.experimental.pallas import tpu as pltpu
from jax.experimental.pallas import tpu_sc as plsc

_TAU = 1.0
_C = 14
_B = 4
_P = 512 * 512
_BLK = 8192

_NSUB = 16
_PER = (_B * _P) // _NSUB
_CHUNK = 16384
_NCH = _PER // _CHUNK
_VECS = _CHUNK // 16


_ROWS = 16


def _kl_body(s_ref, t_ref, out_ref):
    s = s_ref[0]
    t = t_ref[0]
    ms = jnp.max(s, axis=0)
    mt = jnp.max(t, axis=0)
    es = jnp.exp(s - ms[None])
    et = jnp.exp(t - mt[None])
    zs = jnp.sum(es, axis=0)
    zt = jnp.sum(et, axis=0)
    num = jnp.sum(et * (t - s), axis=0)
    out_ref[...] = num / zt + (ms - mt) + jnp.log(zs) - jnp.log(zt)


def _seg_body(kl_hbm, gt_hbm, out_hbm, klv, gtv, acc2, cnt2, svb, cvb,
              outv, sh_sums, sh_cnts):
    sid = lax.axis_index("s")
    base = sid * _PER
    lane = lax.iota(jnp.int32, 16)
    zero16 = jnp.zeros((16,), jnp.float32)
    ones16 = jnp.ones((16,), jnp.float32)

    lane16 = lane * 16
    for r in range(16):
        acc2[pl.ds(r * 16, 16)] = zero16
        cnt2[pl.ds(r * 16, 16)] = zero16

    for ch in range(_NCH):
        start = base + ch * _CHUNK
        pltpu.sync_copy(kl_hbm.at[pl.ds(start, _CHUNK)], klv)
        pltpu.sync_copy(gt_hbm.at[pl.ds(start, _CHUNK)], gtv)

        def body(i, carry):
            off = i * 16
            kv = klv[pl.ds(off, 16)]
            gv = gtv[pl.ds(off, 16)]
            plsc.addupdate_scatter(acc2, [lane16 + gv], kv)
            plsc.addupdate_scatter(cnt2, [lane16 + gv], ones16)
            return carry

        lax.fori_loop(0, _VECS, body, 0)

    sv = acc2[pl.ds(0, 16)]
    cv = cnt2[pl.ds(0, 16)]
    for r in range(1, 16):
        sv = sv + acc2[pl.ds(r * 16, 16)]
        cv = cv + cnt2[pl.ds(r * 16, 16)]
    svb[...] = sv
    cvb[...] = cv

    pltpu.sync_copy(svb, sh_sums.at[pl.ds(sid * 16, 16)])
    pltpu.sync_copy(cvb, sh_cnts.at[pl.ds(sid * 16, 16)])
    plsc.subcore_barrier()

    @pl.when(sid == 0)
    def _():
        pltpu.sync_copy(sh_sums, acc2)
        pltpu.sync_copy(sh_cnts, cnt2)
        cls_mask = (lane >= 1) & (lane < _C)
        spb = _NSUB // _B
        total = jnp.float32(0.0)
        for bb in range(_B):
            s = acc2[pl.ds(spb * bb * 16, 16)]
            c = cnt2[pl.ds(spb * bb * 16, 16)]
            for r in range(1, spb):
                s = s + acc2[pl.ds((spb * bb + r) * 16, 16)]
                c = c + cnt2[pl.ds((spb * bb + r) * 16, 16)]
            per = jnp.where(c > 0.0, s / (_C * jnp.maximum(c, 1.0)), 0.0)
            per = jnp.where(cls_mask, per, 0.0)
            total = total + jnp.sum(per)
        outv[...] = ones16 * total
        pltpu.sync_copy(outv, out_hbm)


@functools.cache
def _seg_reduce():
    @functools.partial(
        pl.kernel,
        out_type=jax.ShapeDtypeStruct((16,), jnp.float32),
        mesh=plsc.VectorSubcoreMesh(core_axis_name="c", subcore_axis_name="s",
                                    num_cores=1),
        compiler_params=pltpu.CompilerParams(needs_layout_passes=False),
        scratch_types=[
            pltpu.VMEM((_CHUNK,), jnp.float32),
            pltpu.VMEM((_CHUNK,), jnp.int32),
            pltpu.VMEM((256,), jnp.float32),
            pltpu.VMEM((256,), jnp.float32),
            pltpu.VMEM((16,), jnp.float32),
            pltpu.VMEM((16,), jnp.float32),
            pltpu.VMEM((16,), jnp.float32),
            pltpu.VMEM_SHARED((_NSUB * 16,), jnp.float32),
            pltpu.VMEM_SHARED((_NSUB * 16,), jnp.float32),
        ],
    )
    def seg(kl_hbm, gt_hbm, out_hbm, *scratch):
        _seg_body(kl_hbm, gt_hbm, out_hbm, *scratch)

    return seg


def kernel(preds_S, preds_T, gt_labels):
    nj = 512 // _ROWS
    kl_pix = pl.pallas_call(
        _kl_body,
        grid=(_B, nj),
        in_specs=[
            pl.BlockSpec((1, _C, _ROWS, 512), lambda b, j: (b, 0, j, 0)),
            pl.BlockSpec((1, _C, _ROWS, 512), lambda b, j: (b, 0, j, 0)),
        ],
        out_specs=pl.BlockSpec((_ROWS, 512), lambda b, j: (b * nj + j, 0)),
        out_shape=jax.ShapeDtypeStruct((_B * 512, 512), jnp.float32),
    )(preds_S, preds_T)

    gt_flat = gt_labels.reshape(_B * _P).astype(jnp.int32)
    out_vec = _seg_reduce()(kl_pix.reshape(_B * _P), gt_flat)
    return out_vec[0] * (_TAU * _TAU)


# speedup vs baseline: 12.2906x; 1.0388x over previous
import functools

import jax
import jax.numpy as jnp
from jax import lax
from jax.experimental import pallas as pl
from jax.experimental.pallas import tpu as pltpu
from jax.experimental.pallas import tpu_sc as plsc

_TAU = 1.0
_C = 14
_B = 4
_P = 512 * 512
_BLK = 8192

_NSUB = 16
_PER = (_B * _P) // _NSUB
_CHUNK = 16384
_NCH = _PER // _CHUNK
_VECS = _CHUNK // 16


_ROWS = 16


def _kl_body(s_ref, t_ref, out_ref):
    s = s_ref[0]
    t = t_ref[0]
    ms = jnp.max(s, axis=0)
    mt = jnp.max(t, axis=0)
    es = jnp.exp(s - ms[None])
    et = jnp.exp(t - mt[None])
    zs = jnp.sum(es, axis=0)
    zt = jnp.sum(et, axis=0)
    num = jnp.sum(et * (t - s), axis=0)
    out_ref[...] = num / zt + (ms - mt) + jnp.log(zs) - jnp.log(zt)


def _seg_body(kl_hbm, gt_hbm, out_hbm, klv, gtv, acc2, cnt2, svb, cvb,
              outv, sh_sums, sh_cnts, sem0, sem1):
    sid = lax.axis_index("s")
    base = sid * _PER
    lane = lax.iota(jnp.int32, 16)
    zero16 = jnp.zeros((16,), jnp.float32)
    ones16 = jnp.ones((16,), jnp.float32)

    lane16 = lane * 16
    for r in range(16):
        acc2[pl.ds(r * 16, 16)] = zero16
        cnt2[pl.ds(r * 16, 16)] = zero16

    sems = (sem0, sem1)

    def mk(ch, buf):
        start = base + ch * _CHUNK
        return (
            pltpu.make_async_copy(kl_hbm.at[pl.ds(start, _CHUNK)],
                                  klv.at[buf], sems[buf]),
            pltpu.make_async_copy(gt_hbm.at[pl.ds(start, _CHUNK)],
                                  gtv.at[buf], sems[buf]),
        )

    pend = {0: mk(0, 0)}
    pend[0][0].start()
    pend[0][1].start()
    for ch in range(_NCH):
        buf = ch % 2
        if ch + 1 < _NCH:
            pend[ch + 1] = mk(ch + 1, (ch + 1) % 2)
            pend[ch + 1][0].start()
            pend[ch + 1][1].start()
        pend[ch][0].wait()
        pend[ch][1].wait()

        def body(i, carry):
            off = i * 16
            kv = klv[buf, pl.ds(off, 16)]
            gv = gtv[buf, pl.ds(off, 16)]
            plsc.addupdate_scatter(acc2, [lane16 + gv], kv)
            plsc.addupdate_scatter(cnt2, [lane16 + gv], ones16)
            return carry

        lax.fori_loop(0, _VECS, body, 0)

    sv = acc2[pl.ds(0, 16)]
    cv = cnt2[pl.ds(0, 16)]
    for r in range(1, 16):
        sv = sv + acc2[pl.ds(r * 16, 16)]
        cv = cv + cnt2[pl.ds(r * 16, 16)]
    svb[...] = sv
    cvb[...] = cv

    pltpu.sync_copy(svb, sh_sums.at[pl.ds(sid * 16, 16)])
    pltpu.sync_copy(cvb, sh_cnts.at[pl.ds(sid * 16, 16)])
    plsc.subcore_barrier()

    @pl.when(sid == 0)
    def _():
        pltpu.sync_copy(sh_sums, acc2)
        pltpu.sync_copy(sh_cnts, cnt2)
        cls_mask = (lane >= 1) & (lane < _C)
        spb = _NSUB // _B
        total = jnp.float32(0.0)
        for bb in range(_B):
            s = acc2[pl.ds(spb * bb * 16, 16)]
            c = cnt2[pl.ds(spb * bb * 16, 16)]
            for r in range(1, spb):
                s = s + acc2[pl.ds((spb * bb + r) * 16, 16)]
                c = c + cnt2[pl.ds((spb * bb + r) * 16, 16)]
            per = jnp.where(c > 0.0, s / (_C * jnp.maximum(c, 1.0)), 0.0)
            per = jnp.where(cls_mask, per, 0.0)
            total = total + jnp.sum(per)
        outv[...] = ones16 * total
        pltpu.sync_copy(outv, out_hbm)


@functools.cache
def _seg_reduce():
    @functools.partial(
        pl.kernel,
        out_type=jax.ShapeDtypeStruct((16,), jnp.float32),
        mesh=plsc.VectorSubcoreMesh(core_axis_name="c", subcore_axis_name="s",
                                    num_cores=1),
        compiler_params=pltpu.CompilerParams(needs_layout_passes=False),
        scratch_types=[
            pltpu.VMEM((2, _CHUNK), jnp.float32),
            pltpu.VMEM((2, _CHUNK), jnp.int32),
            pltpu.VMEM((256,), jnp.float32),
            pltpu.VMEM((256,), jnp.float32),
            pltpu.VMEM((16,), jnp.float32),
            pltpu.VMEM((16,), jnp.float32),
            pltpu.VMEM((16,), jnp.float32),
            pltpu.VMEM_SHARED((_NSUB * 16,), jnp.float32),
            pltpu.VMEM_SHARED((_NSUB * 16,), jnp.float32),
            pltpu.SemaphoreType.DMA,
            pltpu.SemaphoreType.DMA,
        ],
    )
    def seg(kl_hbm, gt_hbm, out_hbm, *scratch):
        _seg_body(kl_hbm, gt_hbm, out_hbm, *scratch)

    return seg


def kernel(preds_S, preds_T, gt_labels):
    nj = 512 // _ROWS
    kl_pix = pl.pallas_call(
        _kl_body,
        grid=(_B, nj),
        in_specs=[
            pl.BlockSpec((1, _C, _ROWS, 512), lambda b, j: (b, 0, j, 0)),
            pl.BlockSpec((1, _C, _ROWS, 512), lambda b, j: (b, 0, j, 0)),
        ],
        out_specs=pl.BlockSpec((_ROWS, 512), lambda b, j: (b * nj + j, 0)),
        out_shape=jax.ShapeDtypeStruct((_B * 512, 512), jnp.float32),
    )(preds_S, preds_T)

    gt_flat = gt_labels.reshape(_B * _P).astype(jnp.int32)
    out_vec = _seg_reduce()(kl_pix.reshape(_B * _P), gt_flat)
    return out_vec[0] * (_TAU * _TAU)


# speedup vs baseline: 12.2994x; 1.0007x over previous
import functools

import jax
import jax.numpy as jnp
from jax import lax
from jax.experimental import pallas as pl
from jax.experimental.pallas import tpu as pltpu
from jax.experimental.pallas import tpu_sc as plsc

_TAU = 1.0
_C = 14
_B = 4
_P = 512 * 512
_BLK = 8192

_NSUB = 16
_PER = (_B * _P) // _NSUB
_CHUNK = 16384
_NCH = _PER // _CHUNK
_VECS = _CHUNK // 16
_UNROLL = 8


_ROWS = 16


def _kl_body(s_ref, t_ref, g_ref, out_ref, gout_ref):
    s = s_ref[0]
    t = t_ref[0]
    ms = jnp.max(s, axis=0)
    mt = jnp.max(t, axis=0)
    es = jnp.exp(s - ms[None])
    et = jnp.exp(t - mt[None])
    zs = jnp.sum(es, axis=0)
    zt = jnp.sum(et, axis=0)
    klp = jnp.sum(et * (t - s), axis=0) / zt \
        + (ms - mt) + jnp.log(zs) - jnp.log(zt)
    out_ref[...] = klp.reshape(_ROWS * 4, 128)
    gout_ref[...] = g_ref[0, 0].reshape(_ROWS * 4, 128)


def _seg_body(kl_hbm, gt_hbm, out_hbm, klv, gtv, acc2, cnt2, svb, cvb,
              outv, sh_sums, sh_cnts, sem0, sem1):
    sid = lax.axis_index("s")
    base = sid * _PER
    lane = lax.iota(jnp.int32, 16)
    zero16 = jnp.zeros((16,), jnp.float32)
    ones16 = jnp.ones((16,), jnp.float32)

    lane16 = lane * 16
    for r in range(16):
        acc2[pl.ds(r * 16, 16)] = zero16
        cnt2[pl.ds(r * 16, 16)] = zero16

    sems = (sem0, sem1)

    def mk(ch, buf):
        start = base + ch * _CHUNK
        return (
            pltpu.make_async_copy(kl_hbm.at[pl.ds(start, _CHUNK)],
                                  klv.at[buf], sems[buf]),
            pltpu.make_async_copy(gt_hbm.at[pl.ds(start, _CHUNK)],
                                  gtv.at[buf], sems[buf]),
        )

    pend = {0: mk(0, 0)}
    pend[0][0].start()
    pend[0][1].start()
    for ch in range(_NCH):
        buf = ch % 2
        if ch + 1 < _NCH:
            pend[ch + 1] = mk(ch + 1, (ch + 1) % 2)
            pend[ch + 1][0].start()
            pend[ch + 1][1].start()
        pend[ch][0].wait()
        pend[ch][1].wait()

        def body(i, carry):
            off = i * (16 * _UNROLL)
            for u in range(_UNROLL):
                kv = klv[buf, pl.ds(off + u * 16, 16)]
                gv = gtv[buf, pl.ds(off + u * 16, 16)]
                plsc.addupdate_scatter(acc2, [lane16 + gv], kv)
                plsc.addupdate_scatter(cnt2, [lane16 + gv], ones16)
            return carry

        lax.fori_loop(0, _VECS // _UNROLL, body, 0)

    sv = acc2[pl.ds(0, 16)]
    cv = cnt2[pl.ds(0, 16)]
    for r in range(1, 16):
        sv = sv + acc2[pl.ds(r * 16, 16)]
        cv = cv + cnt2[pl.ds(r * 16, 16)]
    svb[...] = sv
    cvb[...] = cv

    pltpu.sync_copy(svb, sh_sums.at[pl.ds(sid * 16, 16)])
    pltpu.sync_copy(cvb, sh_cnts.at[pl.ds(sid * 16, 16)])
    plsc.subcore_barrier()

    @pl.when(sid == 0)
    def _():
        pltpu.sync_copy(sh_sums, acc2)
        pltpu.sync_copy(sh_cnts, cnt2)
        cls_mask = (lane >= 1) & (lane < _C)
        spb = _NSUB // _B
        total = jnp.float32(0.0)
        for bb in range(_B):
            s = acc2[pl.ds(spb * bb * 16, 16)]
            c = cnt2[pl.ds(spb * bb * 16, 16)]
            for r in range(1, spb):
                s = s + acc2[pl.ds((spb * bb + r) * 16, 16)]
                c = c + cnt2[pl.ds((spb * bb + r) * 16, 16)]
            per = jnp.where(c > 0.0, s / (_C * jnp.maximum(c, 1.0)), 0.0)
            per = jnp.where(cls_mask, per, 0.0)
            total = total + jnp.sum(per)
        outv[...] = ones16 * total
        pltpu.sync_copy(outv, out_hbm)


@functools.cache
def _seg_reduce():
    @functools.partial(
        pl.kernel,
        out_type=jax.ShapeDtypeStruct((16,), jnp.float32),
        mesh=plsc.VectorSubcoreMesh(core_axis_name="c", subcore_axis_name="s",
                                    num_cores=1),
        compiler_params=pltpu.CompilerParams(needs_layout_passes=False),
        scratch_types=[
            pltpu.VMEM((2, _CHUNK), jnp.float32),
            pltpu.VMEM((2, _CHUNK), jnp.int32),
            pltpu.VMEM((256,), jnp.float32),
            pltpu.VMEM((256,), jnp.float32),
            pltpu.VMEM((16,), jnp.float32),
            pltpu.VMEM((16,), jnp.float32),
            pltpu.VMEM((16,), jnp.float32),
            pltpu.VMEM_SHARED((_NSUB * 16,), jnp.float32),
            pltpu.VMEM_SHARED((_NSUB * 16,), jnp.float32),
            pltpu.SemaphoreType.DMA,
            pltpu.SemaphoreType.DMA,
        ],
    )
    def seg(kl_hbm, gt_hbm, out_hbm, *scratch):
        _seg_body(kl_hbm, gt_hbm, out_hbm, *scratch)

    return seg


def kernel(preds_S, preds_T, gt_labels):
    nj = 512 // _ROWS
    rows = _ROWS * 4
    kl_pix, gt_lin = pl.pallas_call(
        _kl_body,
        grid=(_B, nj),
        in_specs=[
            pl.BlockSpec((1, _C, _ROWS, 512), lambda b, j: (b, 0, j, 0)),
            pl.BlockSpec((1, _C, _ROWS, 512), lambda b, j: (b, 0, j, 0)),
            pl.BlockSpec((1, 1, _ROWS, 512), lambda b, j: (b, 0, j, 0)),
        ],
        out_specs=[
            pl.BlockSpec((rows, 128), lambda b, j: (b * nj + j, 0)),
            pl.BlockSpec((rows, 128), lambda b, j: (b * nj + j, 0)),
        ],
        out_shape=[
            jax.ShapeDtypeStruct((_B * _P // 128, 128), jnp.float32),
            jax.ShapeDtypeStruct((_B * _P // 128, 128), jnp.int32),
        ],
    )(preds_S, preds_T, gt_labels)

    out_vec = _seg_reduce()(kl_pix.reshape(_B * _P), gt_lin.reshape(_B * _P))
    return out_vec[0] * (_TAU * _TAU)


# speedup vs baseline: 15.0867x; 1.2266x over previous
import functools

import jax
import jax.numpy as jnp
from jax import lax
from jax.experimental import pallas as pl
from jax.experimental.pallas import tpu as pltpu
from jax.experimental.pallas import tpu_sc as plsc

_TAU = 1.0
_C = 14
_B = 4
_P = 512 * 512
_BLK = 8192

_NSUB = 16
_PER = (_B * _P) // _NSUB
_CHUNK = 16384
_NCH = _PER // _CHUNK
_VECS = _CHUNK // 16
_UNROLL = 8


_ROWS = 32


def _kl_body(s_ref, t_ref, g_ref, out_ref, gout_ref):
    s = s_ref[0]
    t = t_ref[0]
    ms = jnp.max(s, axis=0)
    mt = jnp.max(t, axis=0)
    es = jnp.exp(s - ms[None])
    et = jnp.exp(t - mt[None])
    zs = jnp.sum(es, axis=0)
    zt = jnp.sum(et, axis=0)
    klp = jnp.sum(et * (t - s), axis=0) / zt \
        + (ms - mt) + jnp.log(zs) - jnp.log(zt)
    out_ref[...] = klp.reshape(_ROWS * 4, 128)
    gout_ref[...] = g_ref[0, 0].reshape(_ROWS * 4, 128)


def _seg_body(kl_hbm, gt_hbm, out_hbm, klv, gtv, acc2, cnt2, svb, cvb,
              outv, sh_sums, sh_cnts, sem0, sem1):
    sid = lax.axis_index("s")
    base = sid * _PER
    lane = lax.iota(jnp.int32, 16)
    zero16 = jnp.zeros((16,), jnp.float32)
    ones16 = jnp.ones((16,), jnp.float32)

    lane16 = lane * 16
    for r in range(16):
        acc2[pl.ds(r * 16, 16)] = zero16
        cnt2[pl.ds(r * 16, 16)] = zero16

    sems = (sem0, sem1)

    def mk(ch, buf):
        start = base + ch * _CHUNK
        return (
            pltpu.make_async_copy(kl_hbm.at[pl.ds(start, _CHUNK)],
                                  klv.at[buf], sems[buf]),
            pltpu.make_async_copy(gt_hbm.at[pl.ds(start, _CHUNK)],
                                  gtv.at[buf], sems[buf]),
        )

    pend = {0: mk(0, 0)}
    pend[0][0].start()
    pend[0][1].start()
    for ch in range(_NCH):
        buf = ch % 2
        if ch + 1 < _NCH:
            pend[ch + 1] = mk(ch + 1, (ch + 1) % 2)
            pend[ch + 1][0].start()
            pend[ch + 1][1].start()
        pend[ch][0].wait()
        pend[ch][1].wait()

        def body(i, carry):
            off = i * (16 * _UNROLL)
            for u in range(_UNROLL):
                kv = klv[buf, pl.ds(off + u * 16, 16)]
                gv = gtv[buf, pl.ds(off + u * 16, 16)]
                plsc.addupdate_scatter(acc2, [lane16 + gv], kv)
                plsc.addupdate_scatter(cnt2, [lane16 + gv], ones16)
            return carry

        lax.fori_loop(0, _VECS // _UNROLL, body, 0)

    sv = acc2[pl.ds(0, 16)]
    cv = cnt2[pl.ds(0, 16)]
    for r in range(1, 16):
        sv = sv + acc2[pl.ds(r * 16, 16)]
        cv = cv + cnt2[pl.ds(r * 16, 16)]
    svb[...] = sv
    cvb[...] = cv

    pltpu.sync_copy(svb, sh_sums.at[pl.ds(sid * 16, 16)])
    pltpu.sync_copy(cvb, sh_cnts.at[pl.ds(sid * 16, 16)])
    plsc.subcore_barrier()

    @pl.when(sid == 0)
    def _():
        pltpu.sync_copy(sh_sums, acc2)
        pltpu.sync_copy(sh_cnts, cnt2)
        cls_mask = (lane >= 1) & (lane < _C)
        spb = _NSUB // _B
        total = jnp.float32(0.0)
        for bb in range(_B):
            s = acc2[pl.ds(spb * bb * 16, 16)]
            c = cnt2[pl.ds(spb * bb * 16, 16)]
            for r in range(1, spb):
                s = s + acc2[pl.ds((spb * bb + r) * 16, 16)]
                c = c + cnt2[pl.ds((spb * bb + r) * 16, 16)]
            per = jnp.where(c > 0.0, s / (_C * jnp.maximum(c, 1.0)), 0.0)
            per = jnp.where(cls_mask, per, 0.0)
            total = total + jnp.sum(per)
        outv[...] = ones16 * total
        pltpu.sync_copy(outv, out_hbm)


@functools.cache
def _seg_reduce():
    @functools.partial(
        pl.kernel,
        out_type=jax.ShapeDtypeStruct((16,), jnp.float32),
        mesh=plsc.VectorSubcoreMesh(core_axis_name="c", subcore_axis_name="s",
                                    num_cores=1),
        compiler_params=pltpu.CompilerParams(needs_layout_passes=False),
        scratch_types=[
            pltpu.VMEM((2, _CHUNK), jnp.float32),
            pltpu.VMEM((2, _CHUNK), jnp.int32),
            pltpu.VMEM((256,), jnp.float32),
            pltpu.VMEM((256,), jnp.float32),
            pltpu.VMEM((16,), jnp.float32),
            pltpu.VMEM((16,), jnp.float32),
            pltpu.VMEM((16,), jnp.float32),
            pltpu.VMEM_SHARED((_NSUB * 16,), jnp.float32),
            pltpu.VMEM_SHARED((_NSUB * 16,), jnp.float32),
            pltpu.SemaphoreType.DMA,
            pltpu.SemaphoreType.DMA,
        ],
    )
    def seg(kl_hbm, gt_hbm, out_hbm, *scratch):
        _seg_body(kl_hbm, gt_hbm, out_hbm, *scratch)

    return seg


def kernel(preds_S, preds_T, gt_labels):
    nj = 512 // _ROWS
    rows = _ROWS * 4
    kl_pix, gt_lin = pl.pallas_call(
        _kl_body,
        grid=(_B, nj),
        in_specs=[
            pl.BlockSpec((1, _C, _ROWS, 512), lambda b, j: (b, 0, j, 0)),
            pl.BlockSpec((1, _C, _ROWS, 512), lambda b, j: (b, 0, j, 0)),
            pl.BlockSpec((1, 1, _ROWS, 512), lambda b, j: (b, 0, j, 0)),
        ],
        out_specs=[
            pl.BlockSpec((rows, 128), lambda b, j: (b * nj + j, 0)),
            pl.BlockSpec((rows, 128), lambda b, j: (b * nj + j, 0)),
        ],
        out_shape=[
            jax.ShapeDtypeStruct((_B * _P // 128, 128), jnp.float32),
            jax.ShapeDtypeStruct((_B * _P // 128, 128), jnp.int32),
        ],
    )(preds_S, preds_T, gt_labels)

    out_vec = _seg_reduce()(kl_pix.reshape(_B * _P), gt_lin.reshape(_B * _P))
    return out_vec[0] * (_TAU * _TAU)


# speedup vs baseline: 17.3885x; 1.1526x over previous
import functools

import jax
import jax.numpy as jnp
from jax import lax
from jax.experimental import pallas as pl
from jax.experimental.pallas import tpu as pltpu
from jax.experimental.pallas import tpu_sc as plsc

_TAU = 1.0
_C = 14
_B = 4
_P = 512 * 512
_BLK = 8192

_NSUB = 16
_PER = (_B * _P) // _NSUB
_CHUNK = 16384
_NCH = _PER // _CHUNK
_VECS = _CHUNK // 16
_UNROLL = 8


_ROWS = 64


def _kl_body(s_ref, t_ref, g_ref, out_ref, gout_ref):
    s = s_ref[0]
    t = t_ref[0]
    ms = jnp.max(s, axis=0)
    mt = jnp.max(t, axis=0)
    es = jnp.exp(s - ms[None])
    et = jnp.exp(t - mt[None])
    zs = jnp.sum(es, axis=0)
    zt = jnp.sum(et, axis=0)
    klp = jnp.sum(et * (t - s), axis=0) / zt \
        + (ms - mt) + jnp.log(zs) - jnp.log(zt)
    out_ref[...] = klp.reshape(_ROWS * 4, 128)
    gout_ref[...] = g_ref[0, 0].reshape(_ROWS * 4, 128)


def _seg_body(kl_hbm, gt_hbm, out_hbm, klv, gtv, acc2, cnt2, svb, cvb,
              outv, sh_sums, sh_cnts, sem0, sem1):
    sid = lax.axis_index("s")
    base = sid * _PER
    lane = lax.iota(jnp.int32, 16)
    zero16 = jnp.zeros((16,), jnp.float32)
    ones16 = jnp.ones((16,), jnp.float32)

    lane16 = lane * 16
    for r in range(16):
        acc2[pl.ds(r * 16, 16)] = zero16
        cnt2[pl.ds(r * 16, 16)] = zero16

    sems = (sem0, sem1)

    def mk(ch, buf):
        start = base + ch * _CHUNK
        return (
            pltpu.make_async_copy(kl_hbm.at[pl.ds(start, _CHUNK)],
                                  klv.at[buf], sems[buf]),
            pltpu.make_async_copy(gt_hbm.at[pl.ds(start, _CHUNK)],
                                  gtv.at[buf], sems[buf]),
        )

    pend = {0: mk(0, 0)}
    pend[0][0].start()
    pend[0][1].start()
    for ch in range(_NCH):
        buf = ch % 2
        if ch + 1 < _NCH:
            pend[ch + 1] = mk(ch + 1, (ch + 1) % 2)
            pend[ch + 1][0].start()
            pend[ch + 1][1].start()
        pend[ch][0].wait()
        pend[ch][1].wait()

        def body(i, carry):
            off = i * (16 * _UNROLL)
            for u in range(_UNROLL):
                kv = klv[buf, pl.ds(off + u * 16, 16)]
                gv = gtv[buf, pl.ds(off + u * 16, 16)]
                plsc.addupdate_scatter(acc2, [lane16 + gv], kv)
                plsc.addupdate_scatter(cnt2, [lane16 + gv], ones16)
            return carry

        lax.fori_loop(0, _VECS // _UNROLL, body, 0)

    sv = acc2[pl.ds(0, 16)]
    cv = cnt2[pl.ds(0, 16)]
    for r in range(1, 16):
        sv = sv + acc2[pl.ds(r * 16, 16)]
        cv = cv + cnt2[pl.ds(r * 16, 16)]
    svb[...] = sv
    cvb[...] = cv

    pltpu.sync_copy(svb, sh_sums.at[pl.ds(sid * 16, 16)])
    pltpu.sync_copy(cvb, sh_cnts.at[pl.ds(sid * 16, 16)])
    plsc.subcore_barrier()

    @pl.when(sid == 0)
    def _():
        pltpu.sync_copy(sh_sums, acc2)
        pltpu.sync_copy(sh_cnts, cnt2)
        cls_mask = (lane >= 1) & (lane < _C)
        spb = _NSUB // _B
        total = jnp.float32(0.0)
        for bb in range(_B):
            s = acc2[pl.ds(spb * bb * 16, 16)]
            c = cnt2[pl.ds(spb * bb * 16, 16)]
            for r in range(1, spb):
                s = s + acc2[pl.ds((spb * bb + r) * 16, 16)]
                c = c + cnt2[pl.ds((spb * bb + r) * 16, 16)]
            per = jnp.where(c > 0.0, s / (_C * jnp.maximum(c, 1.0)), 0.0)
            per = jnp.where(cls_mask, per, 0.0)
            total = total + jnp.sum(per)
        outv[...] = ones16 * total
        pltpu.sync_copy(outv, out_hbm)


@functools.cache
def _seg_reduce():
    @functools.partial(
        pl.kernel,
        out_type=jax.ShapeDtypeStruct((16,), jnp.float32),
        mesh=plsc.VectorSubcoreMesh(core_axis_name="c", subcore_axis_name="s",
                                    num_cores=1),
        compiler_params=pltpu.CompilerParams(needs_layout_passes=False),
        scratch_types=[
            pltpu.VMEM((2, _CHUNK), jnp.float32),
            pltpu.VMEM((2, _CHUNK), jnp.int32),
            pltpu.VMEM((256,), jnp.float32),
            pltpu.VMEM((256,), jnp.float32),
            pltpu.VMEM((16,), jnp.float32),
            pltpu.VMEM((16,), jnp.float32),
            pltpu.VMEM((16,), jnp.float32),
            pltpu.VMEM_SHARED((_NSUB * 16,), jnp.float32),
            pltpu.VMEM_SHARED((_NSUB * 16,), jnp.float32),
            pltpu.SemaphoreType.DMA,
            pltpu.SemaphoreType.DMA,
        ],
    )
    def seg(kl_hbm, gt_hbm, out_hbm, *scratch):
        _seg_body(kl_hbm, gt_hbm, out_hbm, *scratch)

    return seg


def kernel(preds_S, preds_T, gt_labels):
    nj = 512 // _ROWS
    rows = _ROWS * 4
    kl_pix, gt_lin = pl.pallas_call(
        _kl_body,
        grid=(_B, nj),
        in_specs=[
            pl.BlockSpec((1, _C, _ROWS, 512), lambda b, j: (b, 0, j, 0)),
            pl.BlockSpec((1, _C, _ROWS, 512), lambda b, j: (b, 0, j, 0)),
            pl.BlockSpec((1, 1, _ROWS, 512), lambda b, j: (b, 0, j, 0)),
        ],
        out_specs=[
            pl.BlockSpec((rows, 128), lambda b, j: (b * nj + j, 0)),
            pl.BlockSpec((rows, 128), lambda b, j: (b * nj + j, 0)),
        ],
        out_shape=[
            jax.ShapeDtypeStruct((_B * _P // 128, 128), jnp.float32),
            jax.ShapeDtypeStruct((_B * _P // 128, 128), jnp.int32),
        ],
    )(preds_S, preds_T, gt_labels)

    out_vec = _seg_reduce()(kl_pix.reshape(_B * _P), gt_lin.reshape(_B * _P))
    return out_vec[0] * (_TAU * _TAU)


# speedup vs baseline: 18.6796x; 1.0742x over previous
import functools

import jax
import jax.numpy as jnp
from jax import lax
from jax.experimental import pallas as pl
from jax.experimental.pallas import tpu as pltpu
from jax.experimental.pallas import tpu_sc as plsc

_TAU = 1.0
_C = 14
_B = 4
_P = 512 * 512
_BLK = 8192

_NSUB = 16
_PER = (_B * _P) // _NSUB
_CHUNK = 16384
_NCH = _PER // _CHUNK
_VECS = _CHUNK // 16
_UNROLL = 8


_ROWS = 128


def _kl_body(s_ref, t_ref, g_ref, out_ref, gout_ref):
    s = s_ref[0]
    t = t_ref[0]
    ms = jnp.max(s, axis=0)
    mt = jnp.max(t, axis=0)
    es = jnp.exp(s - ms[None])
    et = jnp.exp(t - mt[None])
    zs = jnp.sum(es, axis=0)
    zt = jnp.sum(et, axis=0)
    klp = jnp.sum(et * (t - s), axis=0) / zt \
        + (ms - mt) + jnp.log(zs) - jnp.log(zt)
    out_ref[...] = klp.reshape(_ROWS * 4, 128)
    gout_ref[...] = g_ref[0, 0].reshape(_ROWS * 4, 128)


def _seg_body(kl_hbm, gt_hbm, out_hbm, klv, gtv, acc2, cnt2, svb, cvb,
              outv, sh_sums, sh_cnts, sem0, sem1):
    sid = lax.axis_index("s")
    base = sid * _PER
    lane = lax.iota(jnp.int32, 16)
    zero16 = jnp.zeros((16,), jnp.float32)
    ones16 = jnp.ones((16,), jnp.float32)

    lane16 = lane * 16
    for r in range(16):
        acc2[pl.ds(r * 16, 16)] = zero16
        cnt2[pl.ds(r * 16, 16)] = zero16

    sems = (sem0, sem1)

    def mk(ch, buf):
        start = base + ch * _CHUNK
        return (
            pltpu.make_async_copy(kl_hbm.at[pl.ds(start, _CHUNK)],
                                  klv.at[buf], sems[buf]),
            pltpu.make_async_copy(gt_hbm.at[pl.ds(start, _CHUNK)],
                                  gtv.at[buf], sems[buf]),
        )

    pend = {0: mk(0, 0)}
    pend[0][0].start()
    pend[0][1].start()
    for ch in range(_NCH):
        buf = ch % 2
        if ch + 1 < _NCH:
            pend[ch + 1] = mk(ch + 1, (ch + 1) % 2)
            pend[ch + 1][0].start()
            pend[ch + 1][1].start()
        pend[ch][0].wait()
        pend[ch][1].wait()

        def body(i, carry):
            off = i * (16 * _UNROLL)
            for u in range(_UNROLL):
                kv = klv[buf, pl.ds(off + u * 16, 16)]
                gv = gtv[buf, pl.ds(off + u * 16, 16)]
                plsc.addupdate_scatter(acc2, [lane16 + gv], kv)
                plsc.addupdate_scatter(cnt2, [lane16 + gv], ones16)
            return carry

        lax.fori_loop(0, _VECS // _UNROLL, body, 0)

    sv = acc2[pl.ds(0, 16)]
    cv = cnt2[pl.ds(0, 16)]
    for r in range(1, 16):
        sv = sv + acc2[pl.ds(r * 16, 16)]
        cv = cv + cnt2[pl.ds(r * 16, 16)]
    svb[...] = sv
    cvb[...] = cv

    pltpu.sync_copy(svb, sh_sums.at[pl.ds(sid * 16, 16)])
    pltpu.sync_copy(cvb, sh_cnts.at[pl.ds(sid * 16, 16)])
    plsc.subcore_barrier()

    @pl.when(sid == 0)
    def _():
        pltpu.sync_copy(sh_sums, acc2)
        pltpu.sync_copy(sh_cnts, cnt2)
        cls_mask = (lane >= 1) & (lane < _C)
        spb = _NSUB // _B
        total = jnp.float32(0.0)
        for bb in range(_B):
            s = acc2[pl.ds(spb * bb * 16, 16)]
            c = cnt2[pl.ds(spb * bb * 16, 16)]
            for r in range(1, spb):
                s = s + acc2[pl.ds((spb * bb + r) * 16, 16)]
                c = c + cnt2[pl.ds((spb * bb + r) * 16, 16)]
            per = jnp.where(c > 0.0, s / (_C * jnp.maximum(c, 1.0)), 0.0)
            per = jnp.where(cls_mask, per, 0.0)
            total = total + jnp.sum(per)
        outv[...] = ones16 * total
        pltpu.sync_copy(outv, out_hbm)


@functools.cache
def _seg_reduce():
    @functools.partial(
        pl.kernel,
        out_type=jax.ShapeDtypeStruct((16,), jnp.float32),
        mesh=plsc.VectorSubcoreMesh(core_axis_name="c", subcore_axis_name="s",
                                    num_cores=1),
        compiler_params=pltpu.CompilerParams(needs_layout_passes=False),
        scratch_types=[
            pltpu.VMEM((2, _CHUNK), jnp.float32),
            pltpu.VMEM((2, _CHUNK), jnp.int32),
            pltpu.VMEM((256,), jnp.float32),
            pltpu.VMEM((256,), jnp.float32),
            pltpu.VMEM((16,), jnp.float32),
            pltpu.VMEM((16,), jnp.float32),
            pltpu.VMEM((16,), jnp.float32),
            pltpu.VMEM_SHARED((_NSUB * 16,), jnp.float32),
            pltpu.VMEM_SHARED((_NSUB * 16,), jnp.float32),
            pltpu.SemaphoreType.DMA,
            pltpu.SemaphoreType.DMA,
        ],
    )
    def seg(kl_hbm, gt_hbm, out_hbm, *scratch):
        _seg_body(kl_hbm, gt_hbm, out_hbm, *scratch)

    return seg


def kernel(preds_S, preds_T, gt_labels):
    nj = 512 // _ROWS
    rows = _ROWS * 4
    kl_pix, gt_lin = pl.pallas_call(
        _kl_body,
        grid=(_B, nj),
        in_specs=[
            pl.BlockSpec((1, _C, _ROWS, 512), lambda b, j: (b, 0, j, 0)),
            pl.BlockSpec((1, _C, _ROWS, 512), lambda b, j: (b, 0, j, 0)),
            pl.BlockSpec((1, 1, _ROWS, 512), lambda b, j: (b, 0, j, 0)),
        ],
        out_specs=[
            pl.BlockSpec((rows, 128), lambda b, j: (b * nj + j, 0)),
            pl.BlockSpec((rows, 128), lambda b, j: (b * nj + j, 0)),
        ],
        out_shape=[
            jax.ShapeDtypeStruct((_B * _P // 128, 128), jnp.float32),
            jax.ShapeDtypeStruct((_B * _P // 128, 128), jnp.int32),
        ],
    )(preds_S, preds_T, gt_labels)

    out_vec = _seg_reduce()(kl_pix.reshape(_B * _P), gt_lin.reshape(_B * _P))
    return out_vec[0] * (_TAU * _TAU)


# speedup vs baseline: 21.8594x; 1.1702x over previous
import functools

import jax
import jax.numpy as jnp
from jax import lax
from jax.experimental import pallas as pl
from jax.experimental.pallas import tpu as pltpu
from jax.experimental.pallas import tpu_sc as plsc

_TAU = 1.0
_C = 14
_B = 4
_P = 512 * 512
_BLK = 8192

_NSUB = 16
_BPC = _B // 2
_PER = (_BPC * _P) // _NSUB
_CHUNK = 16384
_NCH = _PER // _CHUNK
_VECS = _CHUNK // 16
_UNROLL = 8


_ROWS = 128


def _kl_body(s_ref, t_ref, g_ref, out_ref, gout_ref):
    s = s_ref[0]
    t = t_ref[0]
    ms = jnp.max(s, axis=0)
    mt = jnp.max(t, axis=0)
    es = jnp.exp(s - ms[None])
    et = jnp.exp(t - mt[None])
    zs = jnp.sum(es, axis=0)
    zt = jnp.sum(et, axis=0)
    klp = jnp.sum(et * (t - s), axis=0) / zt \
        + (ms - mt) + jnp.log(zs) - jnp.log(zt)
    out_ref[...] = klp.reshape(_ROWS * 4, 128)
    gout_ref[...] = g_ref[0, 0].reshape(_ROWS * 4, 128)


def _seg_body(kl_hbm, gt_hbm, out_hbm, klv, gtv, acc2, cnt2, svb, cvb,
              outv, sh_sums, sh_cnts, sem0, sem1):
    cid = lax.axis_index("c")
    sid = lax.axis_index("s")
    base = cid * (_BPC * _P) + sid * _PER
    lane = lax.iota(jnp.int32, 16)
    zero16 = jnp.zeros((16,), jnp.float32)
    ones16 = jnp.ones((16,), jnp.float32)

    lane16 = lane * 16
    for r in range(32):
        acc2[pl.ds(r * 16, 16)] = zero16
        cnt2[pl.ds(r * 16, 16)] = zero16

    sems = (sem0, sem1)

    def mk(ch, buf):
        start = base + ch * _CHUNK
        return (
            pltpu.make_async_copy(kl_hbm.at[pl.ds(start, _CHUNK)],
                                  klv.at[buf], sems[buf]),
            pltpu.make_async_copy(gt_hbm.at[pl.ds(start, _CHUNK)],
                                  gtv.at[buf], sems[buf]),
        )

    pend = {0: mk(0, 0)}
    pend[0][0].start()
    pend[0][1].start()
    for ch in range(_NCH):
        buf = ch % 2
        if ch + 1 < _NCH:
            pend[ch + 1] = mk(ch + 1, (ch + 1) % 2)
            pend[ch + 1][0].start()
            pend[ch + 1][1].start()
        pend[ch][0].wait()
        pend[ch][1].wait()

        def body(i, carry):
            off = i * (16 * _UNROLL)
            for u in range(_UNROLL):
                kv = klv[buf, pl.ds(off + u * 16, 16)]
                gv = gtv[buf, pl.ds(off + u * 16, 16)]
                tbl = (u % 2) * 256
                plsc.addupdate_scatter(acc2, [tbl + lane16 + gv], kv)
                plsc.addupdate_scatter(cnt2, [tbl + lane16 + gv], ones16)
            return carry

        lax.fori_loop(0, _VECS // _UNROLL, body, 0)

    sv = acc2[pl.ds(0, 16)]
    cv = cnt2[pl.ds(0, 16)]
    for r in range(1, 32):
        sv = sv + acc2[pl.ds(r * 16, 16)]
        cv = cv + cnt2[pl.ds(r * 16, 16)]
    svb[...] = sv
    cvb[...] = cv

    pltpu.sync_copy(svb, sh_sums.at[pl.ds(sid * 16, 16)])
    pltpu.sync_copy(cvb, sh_cnts.at[pl.ds(sid * 16, 16)])
    plsc.subcore_barrier()

    @pl.when(sid == 0)
    def _():
        pltpu.sync_copy(sh_sums, acc2.at[pl.ds(0, _NSUB * 16)])
        pltpu.sync_copy(sh_cnts, cnt2.at[pl.ds(0, _NSUB * 16)])
        cls_mask = (lane >= 1) & (lane < _C)
        spb = _NSUB // _BPC
        total = jnp.float32(0.0)
        for bb in range(_BPC):
            s = acc2[pl.ds(spb * bb * 16, 16)]
            c = cnt2[pl.ds(spb * bb * 16, 16)]
            for r in range(1, spb):
                s = s + acc2[pl.ds((spb * bb + r) * 16, 16)]
                c = c + cnt2[pl.ds((spb * bb + r) * 16, 16)]
            per = jnp.where(c > 0.0, s / (_C * jnp.maximum(c, 1.0)), 0.0)
            per = jnp.where(cls_mask, per, 0.0)
            total = total + jnp.sum(per)
        outv[...] = ones16 * total
        pltpu.sync_copy(outv, out_hbm.at[pl.ds(cid * 16, 16)])


@functools.cache
def _seg_reduce():
    @functools.partial(
        pl.kernel,
        out_type=jax.ShapeDtypeStruct((32,), jnp.float32),
        mesh=plsc.VectorSubcoreMesh(core_axis_name="c", subcore_axis_name="s",
                                    num_cores=2),
        compiler_params=pltpu.CompilerParams(needs_layout_passes=False),
        scratch_types=[
            pltpu.VMEM((2, _CHUNK), jnp.float32),
            pltpu.VMEM((2, _CHUNK), jnp.int32),
            pltpu.VMEM((512,), jnp.float32),
            pltpu.VMEM((512,), jnp.float32),
            pltpu.VMEM((16,), jnp.float32),
            pltpu.VMEM((16,), jnp.float32),
            pltpu.VMEM((16,), jnp.float32),
            pltpu.VMEM_SHARED((_NSUB * 16,), jnp.float32),
            pltpu.VMEM_SHARED((_NSUB * 16,), jnp.float32),
            pltpu.SemaphoreType.DMA,
            pltpu.SemaphoreType.DMA,
        ],
    )
    def seg(kl_hbm, gt_hbm, out_hbm, *scratch):
        _seg_body(kl_hbm, gt_hbm, out_hbm, *scratch)

    return seg


def kernel(preds_S, preds_T, gt_labels):
    nj = 512 // _ROWS
    rows = _ROWS * 4
    kl_pix, gt_lin = pl.pallas_call(
        _kl_body,
        grid=(_B, nj),
        in_specs=[
            pl.BlockSpec((1, _C, _ROWS, 512), lambda b, j: (b, 0, j, 0)),
            pl.BlockSpec((1, _C, _ROWS, 512), lambda b, j: (b, 0, j, 0)),
            pl.BlockSpec((1, 1, _ROWS, 512), lambda b, j: (b, 0, j, 0)),
        ],
        out_specs=[
            pl.BlockSpec((rows, 128), lambda b, j: (b * nj + j, 0)),
            pl.BlockSpec((rows, 128), lambda b, j: (b * nj + j, 0)),
        ],
        out_shape=[
            jax.ShapeDtypeStruct((_B * _P // 128, 128), jnp.float32),
            jax.ShapeDtypeStruct((_B * _P // 128, 128), jnp.int32),
        ],
    )(preds_S, preds_T, gt_labels)

    out_vec = _seg_reduce()(kl_pix.reshape(_B * _P), gt_lin.reshape(_B * _P))
    return (out_vec[0] + out_vec[16]) * (_TAU * _TAU)
